# Initial kernel scaffold; baseline (speedup 1.0000x reference)
import jax
import jax.numpy as jnp
from jax.experimental import pallas as pl


def kernel(data, mem, W, linW, linb, target, sample_id):
    raise NotImplementedError("write your pallas kernel here")


# speedup vs baseline: 52.3768x; 52.3768x over previous
import functools

import jax
import jax.numpy as jnp
from jax import lax
from jax.experimental import pallas as pl
from jax.experimental.pallas import tpu as pltpu
from jax.experimental.pallas import tpu_sc as plsc

_NC = 2
_NS = 16
_NW = _NC * _NS
_YPAD = 128


def _fwd_body(U, H, T, BLK, B, CH,
              data_ref, w_all_ref, linw_ref, linb_ref, sid_blk_ref,
              sid_full_ref, y_ref, widx_ref):
    wc = jnp.concatenate(
        [jnp.dot(w_all_ref[u], linw_ref[u * H:(u + 1) * H, :],
                 preferred_element_type=jnp.float32) for u in range(U)],
        axis=0)
    y = jnp.dot(data_ref[...], wc, preferred_element_type=jnp.float32)
    y = y + linb_ref[...][None, :]
    y_ref[...] = jnp.concatenate(
        [y, jnp.zeros((BLK, _YPAD - T), jnp.float32)], axis=1)

    sid_b = sid_blk_ref[...]
    best = jnp.full((BLK,), -1, jnp.int32)
    for c in range(B // CH):
        sid_c = sid_full_ref[pl.ds(c * CH, CH)]
        eq = sid_b[:, None] == sid_c[None, :]
        cand = jnp.where(
            eq, lax.broadcasted_iota(jnp.int32, (BLK, CH), 1) + c * CH, -1)
        best = jnp.maximum(best, jnp.max(cand, axis=1))
    widx_ref[...] = best


def _loss_body(B, T, out_ref, tgt_ref, loss_ref):
    d = out_ref[:, :T] - tgt_ref[...]
    loss_ref[...] = jnp.reshape(jnp.sum(d * d) * (1.0 / (B * T)), (1, 1))


def kernel(data, mem, W, linW, linb, target, sample_id):
    del mem
    U, DIN, H = W.shape
    B = data.shape[0]
    T = linW.shape[1]
    BLK = 512
    CH = 1024
    b_per_w = B // _NW

    y_pad, widx = pl.pallas_call(
        functools.partial(_fwd_body, U, H, T, BLK, B, CH),
        grid=(B // BLK,),
        in_specs=[
            pl.BlockSpec((BLK, U * DIN), lambda i: (i, 0)),
            pl.BlockSpec((U, DIN, H), lambda i: (0, 0, 0)),
            pl.BlockSpec((U * H, T), lambda i: (0, 0)),
            pl.BlockSpec((T,), lambda i: (0,)),
            pl.BlockSpec((BLK,), lambda i: (i,)),
            pl.BlockSpec((B,), lambda i: (0,)),
        ],
        out_specs=[
            pl.BlockSpec((BLK, _YPAD), lambda i: (i, 0)),
            pl.BlockSpec((BLK,), lambda i: (i,)),
        ],
        out_shape=[
            jax.ShapeDtypeStruct((B, _YPAD), jnp.float32),
            jax.ShapeDtypeStruct((B,), jnp.int32),
        ],
    )(data, W, linW, linb, sample_id, sample_id)

    mesh = plsc.VectorSubcoreMesh(core_axis_name="c", subcore_axis_name="s")

    @functools.partial(
        pl.kernel, mesh=mesh,
        out_type=jax.ShapeDtypeStruct((B, _YPAD), jnp.float32),
        scratch_types=[
            pltpu.VMEM((b_per_w,), jnp.int32),
            pltpu.VMEM((b_per_w, _YPAD), jnp.float32),
            pltpu.SemaphoreType.DMA,
        ],
    )
    def _sc_gather(widx_hbm, y_hbm, out_hbm, idx_v, rows_v, sem):
        wid = lax.axis_index("s") * _NC + lax.axis_index("c")
        base = wid * b_per_w
        pltpu.sync_copy(widx_hbm.at[pl.ds(base, b_per_w)], idx_v)
        pltpu.async_copy(y_hbm.at[idx_v], rows_v, sem).wait()
        pltpu.sync_copy(rows_v, out_hbm.at[pl.ds(base, b_per_w)])

    out_pad = _sc_gather(widx, y_pad)

    loss = pl.pallas_call(
        functools.partial(_loss_body, B, T),
        in_specs=[
            pl.BlockSpec((B, _YPAD), lambda: (0, 0)),
            pl.BlockSpec((B, T), lambda: (0, 0)),
        ],
        out_specs=pl.BlockSpec((1, 1), lambda: (0, 0)),
        out_shape=jax.ShapeDtypeStruct((1, 1), jnp.float32),
    )(out_pad, target)

    return out_pad[:, :T], loss[0, 0]
